# baseline (speedup 1.0000x reference)
import numpy as np
import jax
import jax.numpy as jnp
from jax import lax
from jax.experimental import pallas as pl
from jax.experimental.pallas import tpu as pltpu
from jax.experimental.pallas import tpu_sc as plsc

B, V, F, P = 8, 10000, 20000, 16384
FP = 20480
PB = 8
FB = 2048
NCH = FP // FB
FW = FP // 4
PW = P // 4
PSC = 4096
PSW = PSC // 4
_TF_C = np.uint32(0x1BD11BDA)
_TINY = np.float32(np.finfo(np.float32).tiny)


def _tf_rounds(x0, x1, rots):
    for r in rots:
        x0 = x0 + x1
        x1 = (x1 << np.uint32(r)) | (x1 >> np.uint32(32 - r))
        x1 = x0 ^ x1
    return x0, x1


def _threefry_bits(k1, k2, j):
    ks2 = k1 ^ k2 ^ _TF_C
    x0 = jnp.zeros_like(j) + k1
    x1 = j + k2
    ra = (13, 15, 26, 6)
    rb = (17, 29, 16, 24)
    x0, x1 = _tf_rounds(x0, x1, ra)
    x0 = x0 + k2
    x1 = x1 + (ks2 + np.uint32(1))
    x0, x1 = _tf_rounds(x0, x1, rb)
    x0 = x0 + ks2
    x1 = x1 + (k1 + np.uint32(2))
    x0, x1 = _tf_rounds(x0, x1, ra)
    x0 = x0 + k1
    x1 = x1 + (k2 + np.uint32(3))
    x0, x1 = _tf_rounds(x0, x1, rb)
    x0 = x0 + k2
    x1 = x1 + (ks2 + np.uint32(4))
    x0, x1 = _tf_rounds(x0, x1, ra)
    x0 = x0 + ks2
    x1 = x1 + (k1 + np.uint32(5))
    return x0 ^ x1


def _bits_to_unit(bits):
    fb = (bits >> np.uint32(9)) | np.uint32(0x3F800000)
    return lax.bitcast_convert_type(fb, jnp.float32) - np.float32(1.0)


def _g1_body(vx_hbm, vy_hbm, vz_hbm, fidx_hbm, out_hbm,
             xt, yt, zt, f0, f1, f2, o0, o1, o2, o3, o4, o5, o6, o7, o8):
    wid = lax.axis_index("s") * 2 + lax.axis_index("c")
    b = wid >> 2
    seg = (wid & 3) * FW
    pltpu.sync_copy(vx_hbm.at[b], xt)
    pltpu.sync_copy(vy_hbm.at[b], yt)
    pltpu.sync_copy(vz_hbm.at[b], zt)
    pltpu.sync_copy(fidx_hbm.at[b, 0, pl.ds(seg, FW)], f0)
    pltpu.sync_copy(fidx_hbm.at[b, 1, pl.ds(seg, FW)], f1)
    pltpu.sync_copy(fidx_hbm.at[b, 2, pl.ds(seg, FW)], f2)
    outs = ((f0, o0, o1, o2), (f1, o3, o4, o5), (f2, o6, o7, o8))

    def step(i, _):
        s = pl.ds(i * 16, 16)
        for fk, ox, oy, oz in outs:
            vk = fk[s]
            ox[s] = plsc.load_gather(xt, [vk])
            oy[s] = plsc.load_gather(yt, [vk])
            oz[s] = plsc.load_gather(zt, [vk])
        return 0

    lax.fori_loop(0, FW // 16, step, 0)
    for p, o in enumerate((o0, o1, o2, o3, o4, o5, o6, o7, o8)):
        pltpu.sync_copy(o, out_hbm.at[b, p, pl.ds(seg, FW)])


def _g1(vx, vy, vz, fidxP):
    mesh = plsc.VectorSubcoreMesh(core_axis_name="c", subcore_axis_name="s")
    return pl.kernel(
        _g1_body, mesh=mesh,
        compiler_params=pltpu.CompilerParams(use_tc_tiling_on_sc=False, needs_layout_passes=False),
        out_type=jax.ShapeDtypeStruct((B, 9, FP), jnp.float32),
        scratch_types=(
            [pltpu.VMEM((V,), jnp.float32)] * 3
            + [pltpu.VMEM((FW,), jnp.int32)] * 3
            + [pltpu.VMEM((FW,), jnp.float32)] * 9
        ),
    )(vx, vy, vz, fidxP)


def _cross_planar(t):
    ax = t[3:4, :] - t[0:1, :]
    ay = t[4:5, :] - t[1:2, :]
    az = t[5:6, :] - t[2:3, :]
    bx = t[6:7, :] - t[0:1, :]
    by = t[7:8, :] - t[1:2, :]
    bz = t[8:9, :] - t[2:3, :]
    nx = ay * bz - az * by
    ny = az * bx - ax * bz
    nz = ax * by - ay * bx
    return nx, ny, nz


def _t1_body(trit_ref, logits_ref):
    t = trit_ref[0]
    nx, ny, nz = _cross_planar(t)
    norm = jnp.sqrt(nx * nx + ny * ny + nz * nz)
    lg = jnp.log(jnp.maximum(norm * np.float32(0.5), np.float32(1e-30)))
    fb_row = pl.program_id(1) * 2048 + lax.broadcasted_iota(jnp.int32, (1, 2048), 1)
    logits_ref[0] = jnp.where(fb_row < F, lg, np.float32(-1e30))


def _t1(trit):
    return pl.pallas_call(
        _t1_body,
        grid=(B, FP // 2048),
        in_specs=[pl.BlockSpec((1, 9, 2048), lambda bb, i: (bb, 0, i))],
        out_specs=pl.BlockSpec((1, 1, 2048), lambda bb, i: (bb * (FP // 2048) + i, 0, 0)),
        out_shape=jax.ShapeDtypeStruct((B * (FP // 2048), 1, 2048), jnp.float32),
    )(trit).reshape(B, FP)


def _h_body(k1_hbm, k2_hbm, jst_hbm, out_hbm, k1v, k2v, jv0, ra, rb, sem):
    wid = lax.axis_index("s") * 2 + lax.axis_index("c")
    b = wid >> 2
    p0 = (wid & 3) * PSW
    pltpu.sync_copy(k1_hbm.at[b], k1v)
    pltpu.sync_copy(k2_hbm.at[b], k2v)
    pltpu.sync_copy(jst_hbm.at[wid], jv0)
    k1 = k1v[...]
    k2 = k2v[...]

    def row(jv, buf):
        def chunk(i, jv):
            buf[pl.ds(i * 16, 16)] = _threefry_bits(k1, k2, jv)
            return jv + np.uint32(16)

        jv = lax.fori_loop(0, FP // 16, chunk, jv, unroll=4)
        return jv - np.uint32(FP - F)

    def pair(r2, jv):
        jv = row(jv, ra)
        ha = pltpu.async_copy(ra, out_hbm.at[b, p0 + 2 * r2, :], sem)
        jv = row(jv, rb)
        hb = pltpu.async_copy(rb, out_hbm.at[b, p0 + 2 * r2 + 1, :], sem)
        ha.wait()
        hb.wait()
        return jv

    lax.fori_loop(0, PSW // 2, pair, jv0[...])


def _h(k1bc, k2bc, jst):
    mesh = plsc.VectorSubcoreMesh(core_axis_name="c", subcore_axis_name="s")
    return pl.kernel(
        _h_body, mesh=mesh,
        compiler_params=pltpu.CompilerParams(use_tc_tiling_on_sc=False, needs_layout_passes=False),
        out_type=jax.ShapeDtypeStruct((B, PSC, FP), jnp.uint32),
        scratch_types=(
            [pltpu.VMEM((16,), jnp.uint32)] * 3
            + [pltpu.VMEM((FP,), jnp.uint32)] * 2
            + [pltpu.SemaphoreType.DMA]
        ),
    )(k1bc, k2bc, jst)


def _t2b_body(logits_ref, bits_ref, dep_ref, idx_ref):
    del dep_ref
    coliota_i = lax.broadcasted_iota(jnp.int32, (PB, FB), 1)

    def chunk(c, carry):
        m, am = carry
        bits = bits_ref[0, :, c, :]
        u0 = _bits_to_unit(bits)
        u = jnp.maximum(u0, _TINY)
        nlu = jnp.log(u)
        glog = jnp.log(-nlu)
        lg = logits_ref[0, c, :]
        s = lg[None, :] - glog
        upd = s > m
        m = jnp.maximum(m, s)
        am = jnp.where(upd, c, am)
        return m, am

    m0 = jnp.full((PB, FB), -jnp.inf, jnp.float32)
    am0 = jnp.zeros((PB, FB), jnp.int32)
    m, am = lax.fori_loop(0, NCH, chunk, (m0, am0))
    mx = jnp.max(m, axis=1, keepdims=True)
    f_cand = am * np.int32(FB) + coliota_i
    first = jnp.min(jnp.where(m >= mx, f_cand, np.int32(2**31 - 1)),
                    axis=1, keepdims=True)
    idx_ref[0] = first


def _t2b(logits3d, bits4, dep):
    out = pl.pallas_call(
        _t2b_body,
        grid=(B, PSC // PB),
        in_specs=[
            pl.BlockSpec((1, NCH, FB), lambda bb, p: (bb, 0, 0)),
            pl.BlockSpec((1, PB, NCH, FB), lambda bb, p: (bb, p, 0, 0)),
            pl.BlockSpec(memory_space=pltpu.SMEM),
        ],
        out_specs=pl.BlockSpec((1, PB, 1), lambda bb, p: (bb * (PSC // PB) + p, 0, 0)),
        out_shape=jax.ShapeDtypeStruct((B * (PSC // PB), PB, 1), jnp.int32),
    )(logits3d, bits4, dep)
    return out.reshape(B, PSC)


def _t2_body(keys_ref, logits_ref, idx_ref):
    bb = pl.program_id(0)
    p0 = lax.convert_element_type(np.int32(PSC) + pl.program_id(1) * PB, jnp.uint32)
    k1 = keys_ref[bb, 0]
    k2 = keys_ref[bb, 1]
    rowid = lax.broadcasted_iota(jnp.uint32, (PB, 1), 0)
    rowbase = (p0 + rowid) * np.uint32(F)
    coliota = lax.broadcasted_iota(jnp.uint32, (PB, FB), 1)
    coliota_i = lax.broadcasted_iota(jnp.int32, (PB, FB), 1)

    def chunk(c, carry):
        m, am, j = carry
        bits = _threefry_bits(k1, k2, j)
        u0 = _bits_to_unit(bits)
        u = jnp.maximum(u0, _TINY)
        nlu = jnp.log(u)
        glog = jnp.log(-nlu)
        lg = logits_ref[0, c, :]
        s = lg[None, :] - glog
        upd = s > m
        m = jnp.maximum(m, s)
        am = jnp.where(upd, c, am)
        return m, am, j + np.uint32(FB)

    m0 = jnp.full((PB, FB), -jnp.inf, jnp.float32)
    am0 = jnp.zeros((PB, FB), jnp.int32)
    j0 = rowbase + coliota
    m, am, _ = lax.fori_loop(0, NCH, chunk, (m0, am0, j0))
    mx = jnp.max(m, axis=1, keepdims=True)
    f_cand = am * np.int32(FB) + coliota_i
    first = jnp.min(jnp.where(m >= mx, f_cand, np.int32(2**31 - 1)),
                    axis=1, keepdims=True)
    idx_ref[0] = first


def _t2(keys, logits3d):
    npts = P - PSC
    out = pl.pallas_call(
        _t2_body,
        grid=(B, npts // PB),
        in_specs=[
            pl.BlockSpec(memory_space=pltpu.SMEM),
            pl.BlockSpec((1, NCH, FB), lambda bb, p: (bb, 0, 0)),
        ],
        out_specs=pl.BlockSpec((1, PB, 1), lambda bb, p: (bb * (npts // PB) + p, 0, 0)),
        out_shape=jax.ShapeDtypeStruct((B * (npts // PB), PB, 1), jnp.int32),
    )(keys, logits3d)
    return out.reshape(B, npts)


def _g2_body(vx_hbm, vy_hbm, vz_hbm, fidx_hbm, pidx_hbm, out_hbm,
             xt, yt, zt, fa, pidx_v, ox, oy, oz):
    wid = lax.axis_index("s") * 2 + lax.axis_index("c")
    b = wid >> 2
    seg = (wid & 3) * PW
    pltpu.sync_copy(vx_hbm.at[b], xt)
    pltpu.sync_copy(vy_hbm.at[b], yt)
    pltpu.sync_copy(vz_hbm.at[b], zt)
    pltpu.sync_copy(pidx_hbm.at[b, pl.ds(seg, PW)], pidx_v)
    for k in range(3):
        pltpu.sync_copy(fidx_hbm.at[b, k, pl.ds(0, FP)], fa)

        def step(i, _):
            s = pl.ds(i * 16, 16)
            fid = pidx_v[s]
            vk = plsc.load_gather(fa, [fid])
            ox[s] = plsc.load_gather(xt, [vk])
            oy[s] = plsc.load_gather(yt, [vk])
            oz[s] = plsc.load_gather(zt, [vk])
            return 0

        lax.fori_loop(0, PW // 16, step, 0)
        pltpu.sync_copy(ox, out_hbm.at[b, 3 * k + 0, pl.ds(seg, PW)])
        pltpu.sync_copy(oy, out_hbm.at[b, 3 * k + 1, pl.ds(seg, PW)])
        pltpu.sync_copy(oz, out_hbm.at[b, 3 * k + 2, pl.ds(seg, PW)])


def _g2(vx, vy, vz, fidxP, pidx):
    mesh = plsc.VectorSubcoreMesh(core_axis_name="c", subcore_axis_name="s")
    return pl.kernel(
        _g2_body, mesh=mesh,
        compiler_params=pltpu.CompilerParams(use_tc_tiling_on_sc=False, needs_layout_passes=False),
        out_type=jax.ShapeDtypeStruct((B, 9, P), jnp.float32),
        scratch_types=(
            [pltpu.VMEM((V,), jnp.float32)] * 3
            + [pltpu.VMEM((FP,), jnp.int32)]
            + [pltpu.VMEM((PW,), jnp.int32)]
            + [pltpu.VMEM((PW,), jnp.float32)] * 3
        ),
    )(vx, vy, vz, fidxP, pidx)


def _t3_body(keys_ref, stt_ref, pc_ref, sn_ref):
    bb = pl.program_id(0)
    p0 = lax.convert_element_type(pl.program_id(1) * 2048, jnp.uint32)
    j = p0 + lax.broadcasted_iota(jnp.uint32, (1, 2048), 1)
    e1 = _bits_to_unit(_threefry_bits(keys_ref[bb, 0], keys_ref[bb, 1], j))
    e2 = _bits_to_unit(_threefry_bits(keys_ref[bb, 2], keys_ref[bb, 3], j))
    se = jnp.sqrt(e1)
    w1 = np.float32(1.0) - se
    w2 = (np.float32(1.0) - e2) * se
    w3 = e2 * se
    t = stt_ref[0]
    v0 = t[0:3, :]
    v1 = t[3:6, :]
    v2 = t[6:9, :]
    pc_ref[0] = w1 * v0 + w2 * v1 + w3 * v2
    nx, ny, nz = _cross_planar(t)
    inv = np.float32(1.0) / jnp.sqrt(nx * nx + ny * ny + nz * nz)
    sn_ref[0] = jnp.concatenate([nx, ny, nz], axis=0) * inv


def _t3(keys_e, stt):
    return pl.pallas_call(
        _t3_body,
        grid=(B, P // 2048),
        in_specs=[
            pl.BlockSpec(memory_space=pltpu.SMEM),
            pl.BlockSpec((1, 9, 2048), lambda bb, i: (bb, 0, i)),
        ],
        out_specs=[
            pl.BlockSpec((1, 3, 2048), lambda bb, i: (bb, 0, i)),
            pl.BlockSpec((1, 3, 2048), lambda bb, i: (bb, 0, i)),
        ],
        out_shape=[
            jax.ShapeDtypeStruct((B, 3, P), jnp.float32),
            jax.ShapeDtypeStruct((B, 3, P), jnp.float32),
        ],
    )(keys_e, stt)


def kernel(vertices_batch, faces_batch):
    root = jax.random.key(42)
    ks_l, ke_l = [], []
    for b in range(B):
        kb = jax.random.fold_in(root, b)
        ks, ke1, ke2 = jax.random.split(kb, 3)
        ks_l.append(jax.random.key_data(ks))
        ke_l.append(jnp.concatenate([jax.random.key_data(ke1),
                                     jax.random.key_data(ke2)]))
    keys_s = jnp.stack(ks_l)
    keys_e = jnp.stack(ke_l)

    verts = vertices_batch.astype(jnp.float32)
    vx = verts[:, :, 0]
    vy = verts[:, :, 1]
    vz = verts[:, :, 2]
    fidxP = jnp.pad(faces_batch.transpose(0, 2, 1), ((0, 0), (0, 0), (0, FP - F)))

    k1bc = jnp.broadcast_to(keys_s[:, 0:1], (B, 16))
    k2bc = jnp.broadcast_to(keys_s[:, 1:2], (B, 16))
    jst = jnp.asarray(
        (np.arange(32) % 4 * PSW * F)[:, None] + np.arange(16)[None, :],
        dtype=jnp.uint32)

    tri = _g1(vx, vy, vz, fidxP)
    bitsH = _h(k1bc, k2bc, jst)
    logits = _t1(tri)
    logits3 = logits.reshape(B, NCH, FB)
    idx_hi = _t2(keys_s, logits3)
    dep = idx_hi[:, :1]
    idx_lo = _t2b(logits3, bitsH.reshape(B, PSC, NCH, FB), dep)
    idx = jnp.concatenate([idx_lo, idx_hi], axis=1)
    st = _g2(vx, vy, vz, fidxP, idx)
    pc, sn = _t3(keys_e, st)
    return (pc.transpose(0, 2, 1), sn.transpose(0, 2, 1))


# speedup vs baseline: 1.0143x; 1.0143x over previous
import numpy as np
import jax
import jax.numpy as jnp
from jax import lax
from jax.experimental import pallas as pl
from jax.experimental.pallas import tpu as pltpu
from jax.experimental.pallas import tpu_sc as plsc

B, V, F, P = 8, 10000, 20000, 16384
FP = 20480
PB = 8
FB = 2560
NCH = FP // FB
FW = FP // 4
PW = P // 4
PSC = 4096
PSW = PSC // 4
_TF_C = np.uint32(0x1BD11BDA)
_TINY = np.float32(np.finfo(np.float32).tiny)


def _tf_rounds(x0, x1, rots):
    for r in rots:
        x0 = x0 + x1
        x1 = (x1 << np.uint32(r)) | (x1 >> np.uint32(32 - r))
        x1 = x0 ^ x1
    return x0, x1


def _threefry_bits(k1, k2, j):
    ks2 = k1 ^ k2 ^ _TF_C
    x0 = jnp.zeros_like(j) + k1
    x1 = j + k2
    ra = (13, 15, 26, 6)
    rb = (17, 29, 16, 24)
    x0, x1 = _tf_rounds(x0, x1, ra)
    x0 = x0 + k2
    x1 = x1 + (ks2 + np.uint32(1))
    x0, x1 = _tf_rounds(x0, x1, rb)
    x0 = x0 + ks2
    x1 = x1 + (k1 + np.uint32(2))
    x0, x1 = _tf_rounds(x0, x1, ra)
    x0 = x0 + k1
    x1 = x1 + (k2 + np.uint32(3))
    x0, x1 = _tf_rounds(x0, x1, rb)
    x0 = x0 + k2
    x1 = x1 + (ks2 + np.uint32(4))
    x0, x1 = _tf_rounds(x0, x1, ra)
    x0 = x0 + ks2
    x1 = x1 + (k1 + np.uint32(5))
    return x0 ^ x1


def _bits_to_unit(bits):
    fb = (bits >> np.uint32(9)) | np.uint32(0x3F800000)
    return lax.bitcast_convert_type(fb, jnp.float32) - np.float32(1.0)


def _g1_body(vx_hbm, vy_hbm, vz_hbm, fidx_hbm, out_hbm,
             xt, yt, zt, f0, f1, f2, o0, o1, o2, o3, o4, o5, o6, o7, o8):
    wid = lax.axis_index("s") * 2 + lax.axis_index("c")
    b = wid >> 2
    seg = (wid & 3) * FW
    pltpu.sync_copy(vx_hbm.at[b], xt)
    pltpu.sync_copy(vy_hbm.at[b], yt)
    pltpu.sync_copy(vz_hbm.at[b], zt)
    pltpu.sync_copy(fidx_hbm.at[b, 0, pl.ds(seg, FW)], f0)
    pltpu.sync_copy(fidx_hbm.at[b, 1, pl.ds(seg, FW)], f1)
    pltpu.sync_copy(fidx_hbm.at[b, 2, pl.ds(seg, FW)], f2)
    outs = ((f0, o0, o1, o2), (f1, o3, o4, o5), (f2, o6, o7, o8))

    def step(i, _):
        s = pl.ds(i * 16, 16)
        for fk, ox, oy, oz in outs:
            vk = fk[s]
            ox[s] = plsc.load_gather(xt, [vk])
            oy[s] = plsc.load_gather(yt, [vk])
            oz[s] = plsc.load_gather(zt, [vk])
        return 0

    lax.fori_loop(0, FW // 16, step, 0)
    for p, o in enumerate((o0, o1, o2, o3, o4, o5, o6, o7, o8)):
        pltpu.sync_copy(o, out_hbm.at[b, p, pl.ds(seg, FW)])


def _g1(vx, vy, vz, fidxP):
    mesh = plsc.VectorSubcoreMesh(core_axis_name="c", subcore_axis_name="s")
    return pl.kernel(
        _g1_body, mesh=mesh,
        compiler_params=pltpu.CompilerParams(use_tc_tiling_on_sc=False, needs_layout_passes=False),
        out_type=jax.ShapeDtypeStruct((B, 9, FP), jnp.float32),
        scratch_types=(
            [pltpu.VMEM((V,), jnp.float32)] * 3
            + [pltpu.VMEM((FW,), jnp.int32)] * 3
            + [pltpu.VMEM((FW,), jnp.float32)] * 9
        ),
    )(vx, vy, vz, fidxP)


def _cross_planar(t):
    ax = t[3:4, :] - t[0:1, :]
    ay = t[4:5, :] - t[1:2, :]
    az = t[5:6, :] - t[2:3, :]
    bx = t[6:7, :] - t[0:1, :]
    by = t[7:8, :] - t[1:2, :]
    bz = t[8:9, :] - t[2:3, :]
    nx = ay * bz - az * by
    ny = az * bx - ax * bz
    nz = ax * by - ay * bx
    return nx, ny, nz


def _t1_body(trit_ref, logits_ref):
    t = trit_ref[0]
    nx, ny, nz = _cross_planar(t)
    norm = jnp.sqrt(nx * nx + ny * ny + nz * nz)
    lg = jnp.log(jnp.maximum(norm * np.float32(0.5), np.float32(1e-30)))
    fb_row = pl.program_id(1) * 2048 + lax.broadcasted_iota(jnp.int32, (1, 2048), 1)
    logits_ref[0] = jnp.where(fb_row < F, lg, np.float32(-1e30))


def _t1(trit):
    return pl.pallas_call(
        _t1_body,
        grid=(B, FP // 2048),
        in_specs=[pl.BlockSpec((1, 9, 2048), lambda bb, i: (bb, 0, i))],
        out_specs=pl.BlockSpec((1, 1, 2048), lambda bb, i: (bb * (FP // 2048) + i, 0, 0)),
        out_shape=jax.ShapeDtypeStruct((B * (FP // 2048), 1, 2048), jnp.float32),
    )(trit).reshape(B, FP)


def _h_body(k1_hbm, k2_hbm, jst_hbm, out_hbm, k1v, k2v, jv0, ra, rb, sem):
    wid = lax.axis_index("s") * 2 + lax.axis_index("c")
    b = wid >> 2
    p0 = (wid & 3) * PSW
    pltpu.sync_copy(k1_hbm.at[b], k1v)
    pltpu.sync_copy(k2_hbm.at[b], k2v)
    pltpu.sync_copy(jst_hbm.at[wid], jv0)
    k1 = k1v[...]
    k2 = k2v[...]

    def row(jv, buf):
        def chunk(i, jv):
            buf[pl.ds(i * 16, 16)] = _threefry_bits(k1, k2, jv)
            return jv + np.uint32(16)

        jv = lax.fori_loop(0, FP // 16, chunk, jv, unroll=4)
        return jv - np.uint32(FP - F)

    def pair(r2, jv):
        jv = row(jv, ra)
        ha = pltpu.async_copy(ra, out_hbm.at[b, p0 + 2 * r2, :], sem)
        jv = row(jv, rb)
        hb = pltpu.async_copy(rb, out_hbm.at[b, p0 + 2 * r2 + 1, :], sem)
        ha.wait()
        hb.wait()
        return jv

    lax.fori_loop(0, PSW // 2, pair, jv0[...])


def _h(k1bc, k2bc, jst):
    mesh = plsc.VectorSubcoreMesh(core_axis_name="c", subcore_axis_name="s")
    return pl.kernel(
        _h_body, mesh=mesh,
        compiler_params=pltpu.CompilerParams(use_tc_tiling_on_sc=False, needs_layout_passes=False),
        out_type=jax.ShapeDtypeStruct((B, PSC, FP), jnp.uint32),
        scratch_types=(
            [pltpu.VMEM((16,), jnp.uint32)] * 3
            + [pltpu.VMEM((FP,), jnp.uint32)] * 2
            + [pltpu.SemaphoreType.DMA]
        ),
    )(k1bc, k2bc, jst)


def _t2b_body(logits_ref, bits_ref, dep_ref, idx_ref):
    del dep_ref
    coliota_i = lax.broadcasted_iota(jnp.int32, (PB, FB), 1)

    def chunk(c, carry):
        m, am = carry
        bits = bits_ref[0, :, c, :]
        u0 = _bits_to_unit(bits)
        u = jnp.maximum(u0, _TINY)
        nlu = jnp.log(u)
        glog = jnp.log(-nlu)
        lg = logits_ref[0, c, :]
        s = lg[None, :] - glog
        upd = s > m
        m = jnp.maximum(m, s)
        am = jnp.where(upd, c, am)
        return m, am

    m0 = jnp.full((PB, FB), -jnp.inf, jnp.float32)
    am0 = jnp.zeros((PB, FB), jnp.int32)
    m, am = lax.fori_loop(0, NCH, chunk, (m0, am0))
    mx = jnp.max(m, axis=1, keepdims=True)
    f_cand = am * np.int32(FB) + coliota_i
    first = jnp.min(jnp.where(m >= mx, f_cand, np.int32(2**31 - 1)),
                    axis=1, keepdims=True)
    idx_ref[0] = first


def _t2b(logits3d, bits4, dep):
    out = pl.pallas_call(
        _t2b_body,
        grid=(B, PSC // PB),
        in_specs=[
            pl.BlockSpec((1, NCH, FB), lambda bb, p: (bb, 0, 0)),
            pl.BlockSpec((1, PB, NCH, FB), lambda bb, p: (bb, p, 0, 0)),
            pl.BlockSpec(memory_space=pltpu.SMEM),
        ],
        out_specs=pl.BlockSpec((1, PB, 1), lambda bb, p: (bb * (PSC // PB) + p, 0, 0)),
        out_shape=jax.ShapeDtypeStruct((B * (PSC // PB), PB, 1), jnp.int32),
    )(logits3d, bits4, dep)
    return out.reshape(B, PSC)


def _t2_body(keys_ref, logits_ref, idx_ref):
    bb = pl.program_id(0)
    p0 = lax.convert_element_type(np.int32(PSC) + pl.program_id(1) * PB, jnp.uint32)
    k1 = keys_ref[bb, 0]
    k2 = keys_ref[bb, 1]
    rowid = lax.broadcasted_iota(jnp.uint32, (PB, 1), 0)
    rowbase = (p0 + rowid) * np.uint32(F)
    coliota = lax.broadcasted_iota(jnp.uint32, (PB, FB), 1)
    coliota_i = lax.broadcasted_iota(jnp.int32, (PB, FB), 1)

    def chunk(c, carry):
        m, am, j = carry
        bits = _threefry_bits(k1, k2, j)
        u0 = _bits_to_unit(bits)
        u = jnp.maximum(u0, _TINY)
        nlu = jnp.log(u)
        glog = jnp.log(-nlu)
        lg = logits_ref[0, c, :]
        s = lg[None, :] - glog
        upd = s > m
        m = jnp.maximum(m, s)
        am = jnp.where(upd, c, am)
        return m, am, j + np.uint32(FB)

    m0 = jnp.full((PB, FB), -jnp.inf, jnp.float32)
    am0 = jnp.zeros((PB, FB), jnp.int32)
    j0 = rowbase + coliota
    m, am, _ = lax.fori_loop(0, NCH, chunk, (m0, am0, j0))
    mx = jnp.max(m, axis=1, keepdims=True)
    f_cand = am * np.int32(FB) + coliota_i
    first = jnp.min(jnp.where(m >= mx, f_cand, np.int32(2**31 - 1)),
                    axis=1, keepdims=True)
    idx_ref[0] = first


def _t2(keys, logits3d):
    npts = P - PSC
    out = pl.pallas_call(
        _t2_body,
        grid=(B, npts // PB),
        in_specs=[
            pl.BlockSpec(memory_space=pltpu.SMEM),
            pl.BlockSpec((1, NCH, FB), lambda bb, p: (bb, 0, 0)),
        ],
        out_specs=pl.BlockSpec((1, PB, 1), lambda bb, p: (bb * (npts // PB) + p, 0, 0)),
        out_shape=jax.ShapeDtypeStruct((B * (npts // PB), PB, 1), jnp.int32),
    )(keys, logits3d)
    return out.reshape(B, npts)


def _g2_body(vx_hbm, vy_hbm, vz_hbm, fidx_hbm, pidx_hbm, out_hbm,
             xt, yt, zt, fa, pidx_v, ox, oy, oz):
    wid = lax.axis_index("s") * 2 + lax.axis_index("c")
    b = wid >> 2
    seg = (wid & 3) * PW
    pltpu.sync_copy(vx_hbm.at[b], xt)
    pltpu.sync_copy(vy_hbm.at[b], yt)
    pltpu.sync_copy(vz_hbm.at[b], zt)
    pltpu.sync_copy(pidx_hbm.at[b, pl.ds(seg, PW)], pidx_v)
    for k in range(3):
        pltpu.sync_copy(fidx_hbm.at[b, k, pl.ds(0, FP)], fa)

        def step(i, _):
            s = pl.ds(i * 16, 16)
            fid = pidx_v[s]
            vk = plsc.load_gather(fa, [fid])
            ox[s] = plsc.load_gather(xt, [vk])
            oy[s] = plsc.load_gather(yt, [vk])
            oz[s] = plsc.load_gather(zt, [vk])
            return 0

        lax.fori_loop(0, PW // 16, step, 0)
        pltpu.sync_copy(ox, out_hbm.at[b, 3 * k + 0, pl.ds(seg, PW)])
        pltpu.sync_copy(oy, out_hbm.at[b, 3 * k + 1, pl.ds(seg, PW)])
        pltpu.sync_copy(oz, out_hbm.at[b, 3 * k + 2, pl.ds(seg, PW)])


def _g2(vx, vy, vz, fidxP, pidx):
    mesh = plsc.VectorSubcoreMesh(core_axis_name="c", subcore_axis_name="s")
    return pl.kernel(
        _g2_body, mesh=mesh,
        compiler_params=pltpu.CompilerParams(use_tc_tiling_on_sc=False, needs_layout_passes=False),
        out_type=jax.ShapeDtypeStruct((B, 9, P), jnp.float32),
        scratch_types=(
            [pltpu.VMEM((V,), jnp.float32)] * 3
            + [pltpu.VMEM((FP,), jnp.int32)]
            + [pltpu.VMEM((PW,), jnp.int32)]
            + [pltpu.VMEM((PW,), jnp.float32)] * 3
        ),
    )(vx, vy, vz, fidxP, pidx)


def _t3_body(keys_ref, stt_ref, pc_ref, sn_ref):
    bb = pl.program_id(0)
    p0 = lax.convert_element_type(pl.program_id(1) * 2048, jnp.uint32)
    j = p0 + lax.broadcasted_iota(jnp.uint32, (1, 2048), 1)
    e1 = _bits_to_unit(_threefry_bits(keys_ref[bb, 0], keys_ref[bb, 1], j))
    e2 = _bits_to_unit(_threefry_bits(keys_ref[bb, 2], keys_ref[bb, 3], j))
    se = jnp.sqrt(e1)
    w1 = np.float32(1.0) - se
    w2 = (np.float32(1.0) - e2) * se
    w3 = e2 * se
    t = stt_ref[0]
    v0 = t[0:3, :]
    v1 = t[3:6, :]
    v2 = t[6:9, :]
    pc_ref[0] = w1 * v0 + w2 * v1 + w3 * v2
    nx, ny, nz = _cross_planar(t)
    inv = np.float32(1.0) / jnp.sqrt(nx * nx + ny * ny + nz * nz)
    sn_ref[0] = jnp.concatenate([nx, ny, nz], axis=0) * inv


def _t3(keys_e, stt):
    return pl.pallas_call(
        _t3_body,
        grid=(B, P // 2048),
        in_specs=[
            pl.BlockSpec(memory_space=pltpu.SMEM),
            pl.BlockSpec((1, 9, 2048), lambda bb, i: (bb, 0, i)),
        ],
        out_specs=[
            pl.BlockSpec((1, 3, 2048), lambda bb, i: (bb, 0, i)),
            pl.BlockSpec((1, 3, 2048), lambda bb, i: (bb, 0, i)),
        ],
        out_shape=[
            jax.ShapeDtypeStruct((B, 3, P), jnp.float32),
            jax.ShapeDtypeStruct((B, 3, P), jnp.float32),
        ],
    )(keys_e, stt)


def kernel(vertices_batch, faces_batch):
    root = jax.random.key(42)
    ks_l, ke_l = [], []
    for b in range(B):
        kb = jax.random.fold_in(root, b)
        ks, ke1, ke2 = jax.random.split(kb, 3)
        ks_l.append(jax.random.key_data(ks))
        ke_l.append(jnp.concatenate([jax.random.key_data(ke1),
                                     jax.random.key_data(ke2)]))
    keys_s = jnp.stack(ks_l)
    keys_e = jnp.stack(ke_l)

    verts = vertices_batch.astype(jnp.float32)
    vx = verts[:, :, 0]
    vy = verts[:, :, 1]
    vz = verts[:, :, 2]
    fidxP = jnp.pad(faces_batch.transpose(0, 2, 1), ((0, 0), (0, 0), (0, FP - F)))

    k1bc = jnp.broadcast_to(keys_s[:, 0:1], (B, 16))
    k2bc = jnp.broadcast_to(keys_s[:, 1:2], (B, 16))
    jst = jnp.asarray(
        (np.arange(32) % 4 * PSW * F)[:, None] + np.arange(16)[None, :],
        dtype=jnp.uint32)

    tri = _g1(vx, vy, vz, fidxP)
    bitsH = _h(k1bc, k2bc, jst)
    logits = _t1(tri)
    logits3 = logits.reshape(B, NCH, FB)
    idx_hi = _t2(keys_s, logits3)
    dep = idx_hi[:, :1]
    idx_lo = _t2b(logits3, bitsH.reshape(B, PSC, NCH, FB), dep)
    idx = jnp.concatenate([idx_lo, idx_hi], axis=1)
    st = _g2(vx, vy, vz, fidxP, idx)
    pc, sn = _t3(keys_e, st)
    return (pc.transpose(0, 2, 1), sn.transpose(0, 2, 1))


# speedup vs baseline: 1.5272x; 1.5056x over previous
import numpy as np
import jax
import jax.numpy as jnp
from jax import lax
from jax.experimental import pallas as pl
from jax.experimental.pallas import tpu as pltpu
from jax.experimental.pallas import tpu_sc as plsc

B, V, F, P = 8, 10000, 20000, 16384
FP = 20480
PB = 8
FB = 2560
NCH = FP // FB
FW = FP // 4
PW = P // 4
_TF_C = np.uint32(0x1BD11BDA)
_TINY = np.float32(np.finfo(np.float32).tiny)


def _tf_rounds(x0, x1, rots):
    for r in rots:
        x0 = x0 + x1
        x1 = (x1 << np.uint32(r)) | (x1 >> np.uint32(32 - r))
        x1 = x0 ^ x1
    return x0, x1


def _threefry_bits(k1, k2, j):
    ks2 = k1 ^ k2 ^ _TF_C
    x0 = jnp.zeros_like(j) + k1
    x1 = j + k2
    ra = (13, 15, 26, 6)
    rb = (17, 29, 16, 24)
    x0, x1 = _tf_rounds(x0, x1, ra)
    x0 = x0 + k2
    x1 = x1 + (ks2 + np.uint32(1))
    x0, x1 = _tf_rounds(x0, x1, rb)
    x0 = x0 + ks2
    x1 = x1 + (k1 + np.uint32(2))
    x0, x1 = _tf_rounds(x0, x1, ra)
    x0 = x0 + k1
    x1 = x1 + (k2 + np.uint32(3))
    x0, x1 = _tf_rounds(x0, x1, rb)
    x0 = x0 + k2
    x1 = x1 + (ks2 + np.uint32(4))
    x0, x1 = _tf_rounds(x0, x1, ra)
    x0 = x0 + ks2
    x1 = x1 + (k1 + np.uint32(5))
    return x0 ^ x1


def _bits_to_unit(bits):
    fb = (bits >> np.uint32(9)) | np.uint32(0x3F800000)
    return lax.bitcast_convert_type(fb, jnp.float32) - np.float32(1.0)


def _g1_body(vx_hbm, vy_hbm, vz_hbm, fidx_hbm, out_hbm,
             xt, yt, zt, f0, f1, f2, o0, o1, o2, o3, o4, o5, o6, o7, o8):
    wid = lax.axis_index("s") * 2 + lax.axis_index("c")
    b = wid >> 2
    seg = (wid & 3) * FW
    pltpu.sync_copy(vx_hbm.at[b], xt)
    pltpu.sync_copy(vy_hbm.at[b], yt)
    pltpu.sync_copy(vz_hbm.at[b], zt)
    pltpu.sync_copy(fidx_hbm.at[b, 0, pl.ds(seg, FW)], f0)
    pltpu.sync_copy(fidx_hbm.at[b, 1, pl.ds(seg, FW)], f1)
    pltpu.sync_copy(fidx_hbm.at[b, 2, pl.ds(seg, FW)], f2)
    outs = ((f0, o0, o1, o2), (f1, o3, o4, o5), (f2, o6, o7, o8))

    def step(i, _):
        s = pl.ds(i * 16, 16)
        for fk, ox, oy, oz in outs:
            vk = fk[s]
            ox[s] = plsc.load_gather(xt, [vk])
            oy[s] = plsc.load_gather(yt, [vk])
            oz[s] = plsc.load_gather(zt, [vk])
        return 0

    lax.fori_loop(0, FW // 16, step, 0)
    for p, o in enumerate((o0, o1, o2, o3, o4, o5, o6, o7, o8)):
        pltpu.sync_copy(o, out_hbm.at[b, p, pl.ds(seg, FW)])


def _g1(vx, vy, vz, fidxP):
    mesh = plsc.VectorSubcoreMesh(core_axis_name="c", subcore_axis_name="s")
    return pl.kernel(
        _g1_body, mesh=mesh,
        compiler_params=pltpu.CompilerParams(use_tc_tiling_on_sc=False, needs_layout_passes=False),
        out_type=jax.ShapeDtypeStruct((B, 9, FP), jnp.float32),
        scratch_types=(
            [pltpu.VMEM((V,), jnp.float32)] * 3
            + [pltpu.VMEM((FW,), jnp.int32)] * 3
            + [pltpu.VMEM((FW,), jnp.float32)] * 9
        ),
    )(vx, vy, vz, fidxP)


def _cross_planar(t):
    ax = t[3:4, :] - t[0:1, :]
    ay = t[4:5, :] - t[1:2, :]
    az = t[5:6, :] - t[2:3, :]
    bx = t[6:7, :] - t[0:1, :]
    by = t[7:8, :] - t[1:2, :]
    bz = t[8:9, :] - t[2:3, :]
    nx = ay * bz - az * by
    ny = az * bx - ax * bz
    nz = ax * by - ay * bx
    return nx, ny, nz


def _t1_body(trit_ref, logits_ref):
    t = trit_ref[0]
    nx, ny, nz = _cross_planar(t)
    norm = jnp.sqrt(nx * nx + ny * ny + nz * nz)
    lg = jnp.log(jnp.maximum(norm * np.float32(0.5), np.float32(1e-30)))
    fb_row = pl.program_id(1) * 2048 + lax.broadcasted_iota(jnp.int32, (1, 2048), 1)
    logits_ref[0] = jnp.where(fb_row < F, lg, np.float32(-1e30))


def _t1(trit):
    return pl.pallas_call(
        _t1_body,
        grid=(B, FP // 2048),
        in_specs=[pl.BlockSpec((1, 9, 2048), lambda bb, i: (bb, 0, i))],
        out_specs=pl.BlockSpec((1, 1, 2048), lambda bb, i: (bb * (FP // 2048) + i, 0, 0)),
        out_shape=jax.ShapeDtypeStruct((B * (FP // 2048), 1, 2048), jnp.float32),
    )(trit).reshape(B, FP)


def _t2_body(keys_ref, logits_ref, idx_ref):
    bb = pl.program_id(0)
    p0 = lax.convert_element_type(pl.program_id(1) * PB, jnp.uint32)
    k1 = keys_ref[bb, 0]
    k2 = keys_ref[bb, 1]
    rowid = lax.broadcasted_iota(jnp.uint32, (PB, 1), 0)
    rowbase = (p0 + rowid) * np.uint32(F)
    coliota = lax.broadcasted_iota(jnp.uint32, (PB, FB), 1)
    coliota_i = lax.broadcasted_iota(jnp.int32, (PB, FB), 1)

    def chunk(c, carry):
        m, am, j = carry
        bits = _threefry_bits(k1, k2, j)
        u0 = _bits_to_unit(bits)
        u = jnp.maximum(u0, _TINY)
        nlu = jnp.log(u)
        glog = jnp.log(-nlu)
        lg = logits_ref[0, c, :]
        s = lg[None, :] - glog
        upd = s > m
        m = jnp.maximum(m, s)
        am = jnp.where(upd, c, am)
        return m, am, j + np.uint32(FB)

    m0 = jnp.full((PB, FB), -jnp.inf, jnp.float32)
    am0 = jnp.zeros((PB, FB), jnp.int32)
    j0 = rowbase + coliota
    m, am, _ = lax.fori_loop(0, NCH, chunk, (m0, am0, j0))
    mx = jnp.max(m, axis=1, keepdims=True)
    f_cand = am * np.int32(FB) + coliota_i
    first = jnp.min(jnp.where(m >= mx, f_cand, np.int32(2**31 - 1)),
                    axis=1, keepdims=True)
    idx_ref[0] = first


def _t2(keys, logits3d):
    npts = P
    out = pl.pallas_call(
        _t2_body,
        grid=(B, npts // PB),
        in_specs=[
            pl.BlockSpec(memory_space=pltpu.SMEM),
            pl.BlockSpec((1, NCH, FB), lambda bb, p: (bb, 0, 0)),
        ],
        out_specs=pl.BlockSpec((1, PB, 1), lambda bb, p: (bb * (npts // PB) + p, 0, 0)),
        out_shape=jax.ShapeDtypeStruct((B * (npts // PB), PB, 1), jnp.int32),
    )(keys, logits3d)
    return out.reshape(B, npts)


def _g2_body(vx_hbm, vy_hbm, vz_hbm, fidx_hbm, pidx_hbm, out_hbm,
             xt, yt, zt, fa, pidx_v, ox, oy, oz):
    wid = lax.axis_index("s") * 2 + lax.axis_index("c")
    b = wid >> 2
    seg = (wid & 3) * PW
    pltpu.sync_copy(vx_hbm.at[b], xt)
    pltpu.sync_copy(vy_hbm.at[b], yt)
    pltpu.sync_copy(vz_hbm.at[b], zt)
    pltpu.sync_copy(pidx_hbm.at[b, pl.ds(seg, PW)], pidx_v)
    for k in range(3):
        pltpu.sync_copy(fidx_hbm.at[b, k, pl.ds(0, FP)], fa)

        def step(i, _):
            s = pl.ds(i * 16, 16)
            fid = pidx_v[s]
            vk = plsc.load_gather(fa, [fid])
            ox[s] = plsc.load_gather(xt, [vk])
            oy[s] = plsc.load_gather(yt, [vk])
            oz[s] = plsc.load_gather(zt, [vk])
            return 0

        lax.fori_loop(0, PW // 16, step, 0)
        pltpu.sync_copy(ox, out_hbm.at[b, 3 * k + 0, pl.ds(seg, PW)])
        pltpu.sync_copy(oy, out_hbm.at[b, 3 * k + 1, pl.ds(seg, PW)])
        pltpu.sync_copy(oz, out_hbm.at[b, 3 * k + 2, pl.ds(seg, PW)])


def _g2(vx, vy, vz, fidxP, pidx):
    mesh = plsc.VectorSubcoreMesh(core_axis_name="c", subcore_axis_name="s")
    return pl.kernel(
        _g2_body, mesh=mesh,
        compiler_params=pltpu.CompilerParams(use_tc_tiling_on_sc=False, needs_layout_passes=False),
        out_type=jax.ShapeDtypeStruct((B, 9, P), jnp.float32),
        scratch_types=(
            [pltpu.VMEM((V,), jnp.float32)] * 3
            + [pltpu.VMEM((FP,), jnp.int32)]
            + [pltpu.VMEM((PW,), jnp.int32)]
            + [pltpu.VMEM((PW,), jnp.float32)] * 3
        ),
    )(vx, vy, vz, fidxP, pidx)


def _t3_body(keys_ref, stt_ref, pc_ref, sn_ref):
    bb = pl.program_id(0)
    p0 = lax.convert_element_type(pl.program_id(1) * 2048, jnp.uint32)
    j = p0 + lax.broadcasted_iota(jnp.uint32, (1, 2048), 1)
    e1 = _bits_to_unit(_threefry_bits(keys_ref[bb, 0], keys_ref[bb, 1], j))
    e2 = _bits_to_unit(_threefry_bits(keys_ref[bb, 2], keys_ref[bb, 3], j))
    se = jnp.sqrt(e1)
    w1 = np.float32(1.0) - se
    w2 = (np.float32(1.0) - e2) * se
    w3 = e2 * se
    t = stt_ref[0]
    v0 = t[0:3, :]
    v1 = t[3:6, :]
    v2 = t[6:9, :]
    pc_ref[0] = w1 * v0 + w2 * v1 + w3 * v2
    nx, ny, nz = _cross_planar(t)
    inv = np.float32(1.0) / jnp.sqrt(nx * nx + ny * ny + nz * nz)
    sn_ref[0] = jnp.concatenate([nx, ny, nz], axis=0) * inv


def _t3(keys_e, stt):
    return pl.pallas_call(
        _t3_body,
        grid=(B, P // 2048),
        in_specs=[
            pl.BlockSpec(memory_space=pltpu.SMEM),
            pl.BlockSpec((1, 9, 2048), lambda bb, i: (bb, 0, i)),
        ],
        out_specs=[
            pl.BlockSpec((1, 3, 2048), lambda bb, i: (bb, 0, i)),
            pl.BlockSpec((1, 3, 2048), lambda bb, i: (bb, 0, i)),
        ],
        out_shape=[
            jax.ShapeDtypeStruct((B, 3, P), jnp.float32),
            jax.ShapeDtypeStruct((B, 3, P), jnp.float32),
        ],
    )(keys_e, stt)


def kernel(vertices_batch, faces_batch):
    root = jax.random.key(42)
    ks_l, ke_l = [], []
    for b in range(B):
        kb = jax.random.fold_in(root, b)
        ks, ke1, ke2 = jax.random.split(kb, 3)
        ks_l.append(jax.random.key_data(ks))
        ke_l.append(jnp.concatenate([jax.random.key_data(ke1),
                                     jax.random.key_data(ke2)]))
    keys_s = jnp.stack(ks_l)
    keys_e = jnp.stack(ke_l)

    verts = vertices_batch.astype(jnp.float32)
    vx = verts[:, :, 0]
    vy = verts[:, :, 1]
    vz = verts[:, :, 2]
    fidxP = jnp.pad(faces_batch.transpose(0, 2, 1), ((0, 0), (0, 0), (0, FP - F)))

    tri = _g1(vx, vy, vz, fidxP)
    logits = _t1(tri)
    idx = _t2(keys_s, logits.reshape(B, NCH, FB))
    st = _g2(vx, vy, vz, fidxP, idx)
    pc, sn = _t3(keys_e, st)
    return (pc.transpose(0, 2, 1), sn.transpose(0, 2, 1))


# speedup vs baseline: 1.6154x; 1.0578x over previous
import numpy as np
import jax
import jax.numpy as jnp
from jax import lax
from jax.experimental import pallas as pl
from jax.experimental.pallas import tpu as pltpu
from jax.experimental.pallas import tpu_sc as plsc

B, V, F, P = 8, 10000, 20000, 16384
FP = 20480
PB = 8
FB = 2048
NCH = FP // FB
FW = FP // 4
PW = P // 4
_TF_C = np.uint32(0x1BD11BDA)
_TINY = np.float32(np.finfo(np.float32).tiny)


def _tf_rounds(x0, x1, rots):
    for r in rots:
        x0 = x0 + x1
        x1 = (x1 << np.uint32(r)) | (x1 >> np.uint32(32 - r))
        x1 = x0 ^ x1
    return x0, x1


def _threefry_bits(k1, k2, j):
    ks2 = k1 ^ k2 ^ _TF_C
    x0 = jnp.zeros_like(j) + k1
    x1 = j + k2
    ra = (13, 15, 26, 6)
    rb = (17, 29, 16, 24)
    x0, x1 = _tf_rounds(x0, x1, ra)
    x0 = x0 + k2
    x1 = x1 + (ks2 + np.uint32(1))
    x0, x1 = _tf_rounds(x0, x1, rb)
    x0 = x0 + ks2
    x1 = x1 + (k1 + np.uint32(2))
    x0, x1 = _tf_rounds(x0, x1, ra)
    x0 = x0 + k1
    x1 = x1 + (k2 + np.uint32(3))
    x0, x1 = _tf_rounds(x0, x1, rb)
    x0 = x0 + k2
    x1 = x1 + (ks2 + np.uint32(4))
    x0, x1 = _tf_rounds(x0, x1, ra)
    x0 = x0 + ks2
    x1 = x1 + (k1 + np.uint32(5))
    return x0 ^ x1


def _bits_to_unit(bits):
    fb = (bits >> np.uint32(9)) | np.uint32(0x3F800000)
    return lax.bitcast_convert_type(fb, jnp.float32) - np.float32(1.0)


def _g1_body(vx_hbm, vy_hbm, vz_hbm, fidx_hbm, out_hbm,
             xt, yt, zt, f0, f1, f2, o0, o1, o2, o3, o4, o5, o6, o7, o8):
    wid = lax.axis_index("s") * 2 + lax.axis_index("c")
    b = wid >> 2
    seg = (wid & 3) * FW
    pltpu.sync_copy(vx_hbm.at[b], xt)
    pltpu.sync_copy(vy_hbm.at[b], yt)
    pltpu.sync_copy(vz_hbm.at[b], zt)
    pltpu.sync_copy(fidx_hbm.at[b, 0, pl.ds(seg, FW)], f0)
    pltpu.sync_copy(fidx_hbm.at[b, 1, pl.ds(seg, FW)], f1)
    pltpu.sync_copy(fidx_hbm.at[b, 2, pl.ds(seg, FW)], f2)
    outs = ((f0, o0, o1, o2), (f1, o3, o4, o5), (f2, o6, o7, o8))

    def step(i, _):
        s = pl.ds(i * 16, 16)
        for fk, ox, oy, oz in outs:
            vk = fk[s]
            ox[s] = plsc.load_gather(xt, [vk])
            oy[s] = plsc.load_gather(yt, [vk])
            oz[s] = plsc.load_gather(zt, [vk])
        return 0

    lax.fori_loop(0, FW // 16, step, 0)
    for p, o in enumerate((o0, o1, o2, o3, o4, o5, o6, o7, o8)):
        pltpu.sync_copy(o, out_hbm.at[b, p, pl.ds(seg, FW)])


def _g1(vx, vy, vz, fidxP):
    mesh = plsc.VectorSubcoreMesh(core_axis_name="c", subcore_axis_name="s")
    return pl.kernel(
        _g1_body, mesh=mesh,
        compiler_params=pltpu.CompilerParams(use_tc_tiling_on_sc=False, needs_layout_passes=False),
        out_type=jax.ShapeDtypeStruct((B, 9, FP), jnp.float32),
        scratch_types=(
            [pltpu.VMEM((V,), jnp.float32)] * 3
            + [pltpu.VMEM((FW,), jnp.int32)] * 3
            + [pltpu.VMEM((FW,), jnp.float32)] * 9
        ),
    )(vx, vy, vz, fidxP)


def _cross_planar(t):
    ax = t[3:4, :] - t[0:1, :]
    ay = t[4:5, :] - t[1:2, :]
    az = t[5:6, :] - t[2:3, :]
    bx = t[6:7, :] - t[0:1, :]
    by = t[7:8, :] - t[1:2, :]
    bz = t[8:9, :] - t[2:3, :]
    nx = ay * bz - az * by
    ny = az * bx - ax * bz
    nz = ax * by - ay * bx
    return nx, ny, nz


def _t1_body(trit_ref, logits_ref):
    t = trit_ref[0]
    nx, ny, nz = _cross_planar(t)
    norm = jnp.sqrt(nx * nx + ny * ny + nz * nz)
    lg = jnp.log(jnp.maximum(norm * np.float32(0.5), np.float32(1e-30)))
    fb_row = pl.program_id(1) * 2048 + lax.broadcasted_iota(jnp.int32, (1, 2048), 1)
    logits_ref[0] = jnp.where(fb_row < F, lg, np.float32(-1e30))


def _t1(trit):
    return pl.pallas_call(
        _t1_body,
        grid=(B, FP // 2048),
        in_specs=[pl.BlockSpec((1, 9, 2048), lambda bb, i: (bb, 0, i))],
        out_specs=pl.BlockSpec((1, 1, 2048), lambda bb, i: (bb * (FP // 2048) + i, 0, 0)),
        out_shape=jax.ShapeDtypeStruct((B * (FP // 2048), 1, 2048), jnp.float32),
    )(trit).reshape(B, FP)


def _t2_body(keys_ref, logits_ref, idx_ref):
    bb = pl.program_id(0)
    p0 = lax.convert_element_type(pl.program_id(1) * PB, jnp.uint32)
    k1 = keys_ref[bb, 0]
    k2 = keys_ref[bb, 1]
    rowid = lax.broadcasted_iota(jnp.uint32, (PB, 1), 0)
    rowbase = (p0 + rowid) * np.uint32(F)
    coliota = lax.broadcasted_iota(jnp.uint32, (PB, FB), 1)
    coliota_i = lax.broadcasted_iota(jnp.int32, (PB, FB), 1)

    def chunk(c, carry):
        m, am, j = carry
        bits = _threefry_bits(k1, k2, j)
        u0 = _bits_to_unit(bits)
        u = jnp.maximum(u0, _TINY)
        nlu = jnp.log(u)
        glog = jnp.log(-nlu)
        lg = logits_ref[0, c, :]
        s = lg[None, :] - glog
        upd = s > m
        m = jnp.maximum(m, s)
        am = jnp.where(upd, c, am)
        return m, am, j + np.uint32(FB)

    m0 = jnp.full((PB, FB), -jnp.inf, jnp.float32)
    am0 = jnp.zeros((PB, FB), jnp.int32)
    j0 = rowbase + coliota
    m, am, _ = lax.fori_loop(0, NCH, chunk, (m0, am0, j0))
    mx = jnp.max(m, axis=1, keepdims=True)
    f_cand = am * np.int32(FB) + coliota_i
    first = jnp.min(jnp.where(m >= mx, f_cand, np.int32(2**31 - 1)),
                    axis=1, keepdims=True)
    idx_ref[0] = first


def _t2(keys, logits3d):
    npts = P
    out = pl.pallas_call(
        _t2_body,
        grid=(B, npts // PB),
        in_specs=[
            pl.BlockSpec(memory_space=pltpu.SMEM),
            pl.BlockSpec((1, NCH, FB), lambda bb, p: (bb, 0, 0)),
        ],
        out_specs=pl.BlockSpec((1, PB, 1), lambda bb, p: (bb * (npts // PB) + p, 0, 0)),
        out_shape=jax.ShapeDtypeStruct((B * (npts // PB), PB, 1), jnp.int32),
    )(keys, logits3d)
    return out.reshape(B, npts)


def _g2_body(vx_hbm, vy_hbm, vz_hbm, fidx_hbm, pidx_hbm, out_hbm,
             xt, yt, zt, fa, pidx_v, ox, oy, oz):
    wid = lax.axis_index("s") * 2 + lax.axis_index("c")
    b = wid >> 2
    seg = (wid & 3) * PW
    pltpu.sync_copy(vx_hbm.at[b], xt)
    pltpu.sync_copy(vy_hbm.at[b], yt)
    pltpu.sync_copy(vz_hbm.at[b], zt)
    pltpu.sync_copy(pidx_hbm.at[b, pl.ds(seg, PW)], pidx_v)
    for k in range(3):
        pltpu.sync_copy(fidx_hbm.at[b, k, pl.ds(0, FP)], fa)

        def step(i, _):
            s = pl.ds(i * 16, 16)
            fid = pidx_v[s]
            vk = plsc.load_gather(fa, [fid])
            ox[s] = plsc.load_gather(xt, [vk])
            oy[s] = plsc.load_gather(yt, [vk])
            oz[s] = plsc.load_gather(zt, [vk])
            return 0

        lax.fori_loop(0, PW // 16, step, 0)
        pltpu.sync_copy(ox, out_hbm.at[b, 3 * k + 0, pl.ds(seg, PW)])
        pltpu.sync_copy(oy, out_hbm.at[b, 3 * k + 1, pl.ds(seg, PW)])
        pltpu.sync_copy(oz, out_hbm.at[b, 3 * k + 2, pl.ds(seg, PW)])


def _g2(vx, vy, vz, fidxP, pidx):
    mesh = plsc.VectorSubcoreMesh(core_axis_name="c", subcore_axis_name="s")
    return pl.kernel(
        _g2_body, mesh=mesh,
        compiler_params=pltpu.CompilerParams(use_tc_tiling_on_sc=False, needs_layout_passes=False),
        out_type=jax.ShapeDtypeStruct((B, 9, P), jnp.float32),
        scratch_types=(
            [pltpu.VMEM((V,), jnp.float32)] * 3
            + [pltpu.VMEM((FP,), jnp.int32)]
            + [pltpu.VMEM((PW,), jnp.int32)]
            + [pltpu.VMEM((PW,), jnp.float32)] * 3
        ),
    )(vx, vy, vz, fidxP, pidx)


def _t3_body(keys_ref, stt_ref, pc_ref, sn_ref):
    bb = pl.program_id(0)
    p0 = lax.convert_element_type(pl.program_id(1) * 2048, jnp.uint32)
    j = p0 + lax.broadcasted_iota(jnp.uint32, (1, 2048), 1)
    e1 = _bits_to_unit(_threefry_bits(keys_ref[bb, 0], keys_ref[bb, 1], j))
    e2 = _bits_to_unit(_threefry_bits(keys_ref[bb, 2], keys_ref[bb, 3], j))
    se = jnp.sqrt(e1)
    w1 = np.float32(1.0) - se
    w2 = (np.float32(1.0) - e2) * se
    w3 = e2 * se
    t = stt_ref[0]
    v0 = t[0:3, :]
    v1 = t[3:6, :]
    v2 = t[6:9, :]
    pc_ref[0] = w1 * v0 + w2 * v1 + w3 * v2
    nx, ny, nz = _cross_planar(t)
    inv = np.float32(1.0) / jnp.sqrt(nx * nx + ny * ny + nz * nz)
    sn_ref[0] = jnp.concatenate([nx, ny, nz], axis=0) * inv


def _t3(keys_e, stt):
    return pl.pallas_call(
        _t3_body,
        grid=(B, P // 2048),
        in_specs=[
            pl.BlockSpec(memory_space=pltpu.SMEM),
            pl.BlockSpec((1, 9, 2048), lambda bb, i: (bb, 0, i)),
        ],
        out_specs=[
            pl.BlockSpec((1, 3, 2048), lambda bb, i: (bb, 0, i)),
            pl.BlockSpec((1, 3, 2048), lambda bb, i: (bb, 0, i)),
        ],
        out_shape=[
            jax.ShapeDtypeStruct((B, 3, P), jnp.float32),
            jax.ShapeDtypeStruct((B, 3, P), jnp.float32),
        ],
    )(keys_e, stt)


def kernel(vertices_batch, faces_batch):
    root = jax.random.key(42)
    ks_l, ke_l = [], []
    for b in range(B):
        kb = jax.random.fold_in(root, b)
        ks, ke1, ke2 = jax.random.split(kb, 3)
        ks_l.append(jax.random.key_data(ks))
        ke_l.append(jnp.concatenate([jax.random.key_data(ke1),
                                     jax.random.key_data(ke2)]))
    keys_s = jnp.stack(ks_l)
    keys_e = jnp.stack(ke_l)

    verts = vertices_batch.astype(jnp.float32)
    vx = verts[:, :, 0]
    vy = verts[:, :, 1]
    vz = verts[:, :, 2]
    fidxP = jnp.pad(faces_batch.transpose(0, 2, 1), ((0, 0), (0, 0), (0, FP - F)))

    tri = _g1(vx, vy, vz, fidxP)
    logits = _t1(tri)
    idx = _t2(keys_s, logits.reshape(B, NCH, FB))
    st = _g2(vx, vy, vz, fidxP, idx)
    pc, sn = _t3(keys_e, st)
    return (pc.transpose(0, 2, 1), sn.transpose(0, 2, 1))
